# baseline (speedup 1.0000x reference)
import jax
import jax.numpy as jnp
from jax.experimental import pallas as pl
from jax.experimental.pallas import tpu as pltpu

HIDDEN_NODE = 10
HIDDEN_FEATURE = 8
N_IN = 40
BLOCK_B = 4096

_H, _F, _N = HIDDEN_NODE, HIDDEN_FEATURE, N_IN
_HF = _H * _F


def _onehot_T(idx_row, n, e):
    vals = jax.lax.broadcasted_iota(jnp.int32, (n, e), 0)
    return jnp.where(idx_row == vals, 1.0, 0.0).astype(jnp.float32)


def _dot_t(a, b):
    return jax.lax.dot_general(a, b, (((1,), (1,)), ((), ())),
                               preferred_element_type=jnp.float32)


def _dot(a, b):
    return jax.lax.dot_general(a, b, (((1,), (0,)), ((), ())),
                               preferred_element_type=jnp.float32)


def _body(x_ref, z0_ref, y0_ref, wenc_ref, benc_ref, wencroot_ref,
          wpred_ref, bpred_ref, wpredroot_ref, wdec_ref, bdec_ref,
          wdecroot_ref, ei_ref, ew_ref, enc_ref, dec_ref, out_ref,
          g1_ref, g2_ref, c_ref):
    @pl.when(pl.program_id(0) == 0)
    def _assemble():
        f32 = jnp.float32
        ne = enc_ref.shape[1]
        me = ei_ref.shape[1]
        de = dec_ref.shape[1]

        enc_src = _onehot_T(enc_ref[0:1, :], _N, ne)
        enc_dst = _onehot_T(enc_ref[1:2, :], _H, ne)
        E = _dot_t(enc_src, enc_dst)
        ei_src = _onehot_T(ei_ref[0:1, :], _H, me)
        ei_dst = _onehot_T(ei_ref[1:2, :], _H, me)
        MT = _dot_t(ei_src, ei_dst * ew_ref[0:1, :])
        dec_src = _onehot_T(dec_ref[0:1, :], _H, de)
        dec_dst = _onehot_T(dec_ref[1:2, :], _N, de)
        DT = _dot_t(dec_src, dec_dst)

        rowsel_node = jnp.where(
            jax.lax.broadcasted_iota(jnp.int32, (_HF, _H), 0) // _F
            == jax.lax.broadcasted_iota(jnp.int32, (_HF, _H), 1),
            1.0, 0.0).astype(f32)
        rowsel_feat = jnp.where(
            jax.lax.broadcasted_iota(jnp.int32, (_HF, _F), 0) % _F
            == jax.lax.broadcasted_iota(jnp.int32, (_HF, _F), 1),
            1.0, 0.0).astype(f32)
        blockdiag = jnp.where(
            jax.lax.broadcasted_iota(jnp.int32, (_HF, _HF), 0) // _F
            == jax.lax.broadcasted_iota(jnp.int32, (_HF, _HF), 1) // _F,
            1.0, 0.0).astype(f32)

        wenc_tile = _dot_t(wenc_ref[...], rowsel_feat)
        E80 = _dot_t(E, rowsel_node) * wenc_tile
        R1 = _dot_t(_dot(rowsel_feat, wencroot_ref[...]),
                    rowsel_feat) * blockdiag
        mexp = _dot_t(_dot(rowsel_node, MT), rowsel_node)
        wpred_tile = _dot_t(_dot(rowsel_feat, wpred_ref[...]), rowsel_feat)
        wpredroot_tile = _dot_t(_dot(rowsel_feat, wpredroot_ref[...]),
                                rowsel_feat)
        P2 = mexp * wpred_tile + blockdiag * wpredroot_tile
        wdec_col = _dot(rowsel_feat, wdec_ref[...])
        Q = _dot(rowsel_node, DT) * wdec_col

        P2Q = _dot(P2, Q)
        g1_ref[...] = _dot(E80, P2Q)
        g2_ref[...] = _dot(R1, P2Q)
        b1 = _dot_t(benc_ref[...], rowsel_feat)
        b2 = _dot_t(bpred_ref[...], rowsel_feat)
        c_ref[...] = (_dot(b1, P2Q) + _dot(b2, Q)
                      + bdec_ref[0, 0])

    y = (jnp.dot(x_ref[...], g1_ref[...], preferred_element_type=jnp.float32)
         + jnp.dot(z0_ref[...], g2_ref[...],
                   preferred_element_type=jnp.float32))
    out_ref[...] = y + c_ref[...] + wdecroot_ref[0, 0] * y0_ref[...]


@jax.jit
def kernel(x, z_init, y_init, W_enc_rel, b_enc_rel, W_enc_root, W_pred_rel,
           b_pred_rel, W_pred_root, W_dec_rel, b_dec_rel, W_dec_root,
           edge_index, edge_weight, enc_index, dec_index):
    B = x.shape[0]
    z0f = z_init.reshape(B, _HF)
    y0f = y_init.reshape(B, _N)

    grid = (B // BLOCK_B,)
    full = lambda shape: pl.BlockSpec(shape, lambda i: (0,) * len(shape))

    out = pl.pallas_call(
        _body,
        grid=grid,
        in_specs=[
            pl.BlockSpec((BLOCK_B, _N), lambda i: (i, 0)),
            pl.BlockSpec((BLOCK_B, _HF), lambda i: (i, 0)),
            pl.BlockSpec((BLOCK_B, _N), lambda i: (i, 0)),
            full((1, _F)),
            full((1, _F)),
            full((_F, _F)),
            full((_F, _F)),
            full((1, _F)),
            full((_F, _F)),
            full((_F, 1)),
            full((1, 1)),
            full((1, 1)),
            full((2, edge_index.shape[1])),
            full((1, edge_weight.shape[0])),
            full((2, enc_index.shape[1])),
            full((2, dec_index.shape[1])),
        ],
        out_specs=pl.BlockSpec((BLOCK_B, _N), lambda i: (i, 0)),
        out_shape=jax.ShapeDtypeStruct((B, _N), jnp.float32),
        scratch_shapes=[
            pltpu.VMEM((_N, _N), jnp.float32),
            pltpu.VMEM((_HF, _N), jnp.float32),
            pltpu.VMEM((1, _N), jnp.float32),
        ],
        compiler_params=pltpu.CompilerParams(
            dimension_semantics=("arbitrary",)),
    )(x, z0f, y0f, W_enc_rel, b_enc_rel[None, :], W_enc_root, W_pred_rel,
      b_pred_rel[None, :], W_pred_root, W_dec_rel, b_dec_rel[None, :],
      W_dec_root, edge_index, edge_weight[None, :], enc_index, dec_index)
    return out


# speedup vs baseline: 1.0540x; 1.0540x over previous
import jax
import jax.numpy as jnp
from jax.experimental import pallas as pl
from jax.experimental.pallas import tpu as pltpu

HIDDEN_NODE = 10
HIDDEN_FEATURE = 8
N_IN = 40
BLOCK_B = 8192

_H, _F, _N = HIDDEN_NODE, HIDDEN_FEATURE, N_IN
_HF = _H * _F


def _onehot_T(idx_row, n, e):
    vals = jax.lax.broadcasted_iota(jnp.int32, (n, e), 0)
    return jnp.where(idx_row == vals, 1.0, 0.0).astype(jnp.float32)


def _dot_t(a, b):
    return jax.lax.dot_general(a, b, (((1,), (1,)), ((), ())),
                               preferred_element_type=jnp.float32)


def _dot(a, b):
    return jax.lax.dot_general(a, b, (((1,), (0,)), ((), ())),
                               preferred_element_type=jnp.float32)


def _body(x_ref, z0_ref, y0_ref, wenc_ref, benc_ref, wencroot_ref,
          wpred_ref, bpred_ref, wpredroot_ref, wdec_ref, bdec_ref,
          wdecroot_ref, ei_ref, ew_ref, enc_ref, dec_ref, out_ref,
          g1_ref, g2_ref, c_ref):
    @pl.when(pl.program_id(0) == 0)
    def _assemble():
        f32 = jnp.float32
        ne = enc_ref.shape[1]
        me = ei_ref.shape[1]
        de = dec_ref.shape[1]

        enc_src = _onehot_T(enc_ref[0:1, :], _N, ne)
        enc_dst = _onehot_T(enc_ref[1:2, :], _H, ne)
        E = _dot_t(enc_src, enc_dst)
        ei_src = _onehot_T(ei_ref[0:1, :], _H, me)
        ei_dst = _onehot_T(ei_ref[1:2, :], _H, me)
        MT = _dot_t(ei_src, ei_dst * ew_ref[0:1, :])
        dec_src = _onehot_T(dec_ref[0:1, :], _H, de)
        dec_dst = _onehot_T(dec_ref[1:2, :], _N, de)
        DT = _dot_t(dec_src, dec_dst)

        rowsel_node = jnp.where(
            jax.lax.broadcasted_iota(jnp.int32, (_HF, _H), 0) // _F
            == jax.lax.broadcasted_iota(jnp.int32, (_HF, _H), 1),
            1.0, 0.0).astype(f32)
        rowsel_feat = jnp.where(
            jax.lax.broadcasted_iota(jnp.int32, (_HF, _F), 0) % _F
            == jax.lax.broadcasted_iota(jnp.int32, (_HF, _F), 1),
            1.0, 0.0).astype(f32)
        blockdiag = jnp.where(
            jax.lax.broadcasted_iota(jnp.int32, (_HF, _HF), 0) // _F
            == jax.lax.broadcasted_iota(jnp.int32, (_HF, _HF), 1) // _F,
            1.0, 0.0).astype(f32)

        wenc_tile = _dot_t(wenc_ref[...], rowsel_feat)
        E80 = _dot_t(E, rowsel_node) * wenc_tile
        R1 = _dot_t(_dot(rowsel_feat, wencroot_ref[...]),
                    rowsel_feat) * blockdiag
        mexp = _dot_t(_dot(rowsel_node, MT), rowsel_node)
        wpred_tile = _dot_t(_dot(rowsel_feat, wpred_ref[...]), rowsel_feat)
        wpredroot_tile = _dot_t(_dot(rowsel_feat, wpredroot_ref[...]),
                                rowsel_feat)
        P2 = mexp * wpred_tile + blockdiag * wpredroot_tile
        wdec_col = _dot(rowsel_feat, wdec_ref[...])
        Q = _dot(rowsel_node, DT) * wdec_col

        P2Q = _dot(P2, Q)
        g1_ref[...] = _dot(E80, P2Q)
        g2_ref[...] = _dot(R1, P2Q)
        b1 = _dot_t(benc_ref[...], rowsel_feat)
        b2 = _dot_t(bpred_ref[...], rowsel_feat)
        c_ref[...] = (_dot(b1, P2Q) + _dot(b2, Q)
                      + bdec_ref[0, 0])

    y = (jnp.dot(x_ref[...], g1_ref[...], preferred_element_type=jnp.float32)
         + jnp.dot(z0_ref[...], g2_ref[...],
                   preferred_element_type=jnp.float32))
    out_ref[...] = y + c_ref[...] + wdecroot_ref[0, 0] * y0_ref[...]


@jax.jit
def kernel(x, z_init, y_init, W_enc_rel, b_enc_rel, W_enc_root, W_pred_rel,
           b_pred_rel, W_pred_root, W_dec_rel, b_dec_rel, W_dec_root,
           edge_index, edge_weight, enc_index, dec_index):
    B = x.shape[0]
    z0f = z_init.reshape(B, _HF)
    y0f = y_init.reshape(B, _N)

    grid = (B // BLOCK_B,)
    full = lambda shape: pl.BlockSpec(shape, lambda i: (0,) * len(shape))

    out = pl.pallas_call(
        _body,
        grid=grid,
        in_specs=[
            pl.BlockSpec((BLOCK_B, _N), lambda i: (i, 0)),
            pl.BlockSpec((BLOCK_B, _HF), lambda i: (i, 0)),
            pl.BlockSpec((BLOCK_B, _N), lambda i: (i, 0)),
            full((1, _F)),
            full((1, _F)),
            full((_F, _F)),
            full((_F, _F)),
            full((1, _F)),
            full((_F, _F)),
            full((_F, 1)),
            full((1, 1)),
            full((1, 1)),
            full((2, edge_index.shape[1])),
            full((1, edge_weight.shape[0])),
            full((2, enc_index.shape[1])),
            full((2, dec_index.shape[1])),
        ],
        out_specs=pl.BlockSpec((BLOCK_B, _N), lambda i: (i, 0)),
        out_shape=jax.ShapeDtypeStruct((B, _N), jnp.float32),
        scratch_shapes=[
            pltpu.VMEM((_N, _N), jnp.float32),
            pltpu.VMEM((_HF, _N), jnp.float32),
            pltpu.VMEM((1, _N), jnp.float32),
        ],
        compiler_params=pltpu.CompilerParams(
            dimension_semantics=("arbitrary",)),
    )(x, z0f, y0f, W_enc_rel, b_enc_rel[None, :], W_enc_root, W_pred_rel,
      b_pred_rel[None, :], W_pred_root, W_dec_rel, b_dec_rel[None, :],
      W_dec_root, edge_index, edge_weight[None, :], enc_index, dec_index)
    return out
